# baseline (speedup 1.0000x reference)
import functools

import jax
import jax.numpy as jnp
from jax import lax
from jax.experimental import pallas as pl
from jax.experimental.pallas import tpu as pltpu
from jax.experimental.pallas import tpu_sc as plsc

_D = 128
_NW = 32
_CHUNK = 128
_NCHUNK = 8
_NBUF = 4
_EPS = 1e-8


def _sc_gather_body(pidx_hbm, sidx_hbm, table_hbm, qrows_hbm, srows_hbm,
                    *scratch):
    idx_bufs = scratch[0:_NCHUNK]
    row_bufs = scratch[_NCHUNK:_NCHUNK + _NBUF]
    sidx_v = scratch[_NCHUNK + _NBUF]
    sup_v = scratch[_NCHUNK + _NBUF + 1]
    gsems = scratch[_NCHUNK + _NBUF + 2]
    wsems = scratch[_NCHUNK + _NBUF + 3]
    sem_s = scratch[_NCHUNK + _NBUF + 4]
    wid = lax.axis_index("s") * 2 + lax.axis_index("c")
    base = wid * (_NCHUNK * _CHUNK)
    for j in range(_NCHUNK):
        pltpu.sync_copy(pidx_hbm.at[pl.ds(base + j * _CHUNK, _CHUNK)],
                        idx_bufs[j])
    gh = [None] * _NCHUNK
    wh = [None] * _NCHUNK
    for j in range(_NBUF):
        gh[j] = pltpu.async_copy(table_hbm.at[idx_bufs[j]],
                                 row_bufs[j], gsems.at[j])
    for j in range(_NCHUNK):
        gh[j].wait()
        wh[j] = pltpu.async_copy(
            row_bufs[j % _NBUF],
            qrows_hbm.at[pl.ds(base + j * _CHUNK, _CHUNK)], wsems.at[j % _NBUF])
        if j + _NBUF < _NCHUNK:
            wh[j].wait()
            gh[j + _NBUF] = pltpu.async_copy(
                table_hbm.at[idx_bufs[j + _NBUF]],
                row_bufs[j % _NBUF], gsems.at[j % _NBUF])
    for j in range(_NCHUNK - _NBUF, _NCHUNK):
        wh[j].wait()

    @pl.when(wid == 0)
    def _():
        pltpu.sync_copy(sidx_hbm, sidx_v)
        pltpu.async_copy(table_hbm.at[sidx_v], sup_v, sem_s).wait()
        pltpu.sync_copy(sup_v, srows_hbm)


def _tc_reduce_body(xq_ref, h0_ref, h1_ref, sup_ref, spb_ref, out_ref):
    sup = sup_ref[...]
    spb = spb_ref[...]
    e = sup[:, :64] * (1.0 - spb[:, :64]) + sup[:, 64:] * spb[:, :64]
    tid = lax.broadcasted_iota(jnp.int32, e.shape, 0)
    valid0 = (tid % 2 == 0) & (tid < 10)
    valid1 = (tid % 2 == 1) & (tid < 10)
    m0 = jnp.sum(jnp.where(valid0, e, 0.0), axis=0, keepdims=True) * 0.2
    m1 = jnp.sum(jnp.where(valid1, e, 0.0), axis=0, keepdims=True) * 0.2
    n2 = jnp.maximum(jnp.sqrt(jnp.sum(m0 * m0) + jnp.sum(m1 * m1)), _EPS)
    mm = jnp.concatenate([m0, m0, m1, m1], axis=1)

    x = xq_ref[...]
    lane = lax.broadcasted_iota(jnp.int32, x.shape, 2)
    hsel = jnp.where(lane < 128, h0_ref[...][..., None], h1_ref[...][..., None])
    sel = (hsel == (lane // 64) % 2).astype(jnp.float32)
    num = jnp.sum(x * (mm[None] * sel), axis=2)
    sq = jnp.sum(x * x * sel, axis=2)
    n1 = jnp.maximum(jnp.sqrt(sq), _EPS)
    out_ref[...] = num / (n1 * n2)


def kernel(query, support, symbol_emb):
    b = query.shape[0]
    n_rows = b * 2
    table2 = symbol_emb.reshape(-1, _D)
    qflat = query.reshape(-1).astype(jnp.int32)
    pidx = qflat // 2
    h0 = (query[:, 0].astype(jnp.int32) % 2).reshape(b // 128, 128)
    h1 = (query[:, 1].astype(jnp.int32) % 2).reshape(b // 128, 128)
    sflat = support.reshape(-1).astype(jnp.int32)
    sidx = jnp.concatenate([sflat // 2, jnp.zeros((6,), jnp.int32)])
    spb = jnp.broadcast_to(
        jnp.concatenate([(sflat % 2).astype(jnp.float32),
                         jnp.zeros((6,), jnp.float32)])[:, None],
        (16, _D))

    gather = functools.partial(
        pl.kernel,
        mesh=plsc.VectorSubcoreMesh(core_axis_name="c", subcore_axis_name="s",
                                    num_cores=2),
        out_type=(jax.ShapeDtypeStruct((n_rows, _D), jnp.float32),
                  jax.ShapeDtypeStruct((16, _D), jnp.float32)),
        scratch_types=tuple(
            [pltpu.VMEM((_CHUNK,), jnp.int32)] * _NCHUNK
            + [pltpu.VMEM((_CHUNK, _D), jnp.float32)] * _NBUF
            + [pltpu.VMEM((16,), jnp.int32),
               pltpu.VMEM((16, _D), jnp.float32),
               pltpu.SemaphoreType.DMA((_NBUF,)),
               pltpu.SemaphoreType.DMA((_NBUF,)),
               pltpu.SemaphoreType.DMA]),
    )(_sc_gather_body)
    qrows, srows = gather(pidx, sidx, table2)

    xq = qrows.reshape(b // 128, 128, 2 * _D)
    bq = 16
    out2d = pl.pallas_call(
        _tc_reduce_body,
        grid=(b // 128 // bq,),
        in_specs=[pl.BlockSpec((bq, 128, 2 * _D), lambda i: (i, 0, 0)),
                  pl.BlockSpec((bq, 128), lambda i: (i, 0)),
                  pl.BlockSpec((bq, 128), lambda i: (i, 0)),
                  pl.BlockSpec((16, _D), lambda i: (0, 0)),
                  pl.BlockSpec((16, _D), lambda i: (0, 0))],
        out_specs=pl.BlockSpec((bq, 128), lambda i: (i, 0)),
        out_shape=jax.ShapeDtypeStruct((b // 128, 128), jnp.float32),
    )(xq, h0, h1, srows, spb)
    return out2d.reshape(b)


# speedup vs baseline: 3.4076x; 3.4076x over previous
import functools

import jax
import jax.numpy as jnp
from jax import lax
from jax.experimental import pallas as pl
from jax.experimental.pallas import tpu as pltpu
from jax.experimental.pallas import tpu_sc as plsc

_NW = 32
_CHUNK = 128
_BC = 8192
_EPS = 1e-8


def _k1_support_body(sref, tab_ref, out_ref):
    t = pl.program_id(0)
    c = sref[t] % 128
    x = tab_ref[...]
    lane = lax.broadcasted_iota(jnp.int32, x.shape, 1)
    val = jnp.sum(jnp.where(lane == c, x, 0.0), axis=1,
                  keepdims=True)

    @pl.when(t == 0)
    def _():
        out_ref[...] = jnp.zeros_like(out_ref)

    col = lax.broadcasted_iota(jnp.int32, out_ref.shape, 1)
    out_ref[...] = out_ref[...] + jnp.where(col == t, val, 0.0)


def _support_means(sup):
    t = lax.broadcasted_iota(jnp.int32, sup.shape, 1)
    m0 = jnp.sum(jnp.where((t % 2 == 0) & (t < 10), sup, 0.0), axis=1,
                 keepdims=True) * 0.2
    m1 = jnp.sum(jnp.where((t % 2 == 1) & (t < 10), sup, 0.0), axis=1,
                 keepdims=True) * 0.2
    return m0, m1


def _k2_dense_body(tab_ref, sup_ref, w0_ref, w1_ref, s_ref):
    m0, m1 = _support_means(sup_ref[...])
    x = tab_ref[...]
    w0_ref[...] = jnp.sum(x * m0, axis=0)
    w1_ref[...] = jnp.sum(x * m1, axis=0)
    s_ref[...] = jnp.sum(x * x, axis=0)


def _k3_gather_body(q0_hbm, q1_hbm, w0_hbm, w1_hbm, s_hbm,
                    ga_hbm, gb_hbm, gc_hbm, gd_hbm, *scratch):
    i0b = scratch[0:4]
    i1b = scratch[4:8]
    ga_v, gb_v, gc_v, gd_v, sem = scratch[8:]
    wid = lax.axis_index("s") * 2 + lax.axis_index("c")
    base = wid * 512
    for k in range(4):
        pltpu.sync_copy(q0_hbm.at[pl.ds(base + k * _CHUNK, _CHUNK)], i0b[k])
        pltpu.sync_copy(q1_hbm.at[pl.ds(base + k * _CHUNK, _CHUNK)], i1b[k])
    copies = []
    for k in range(4):
        d = pl.ds(k * _CHUNK, _CHUNK)
        copies.append(pltpu.async_copy(w0_hbm.at[i0b[k]], ga_v.at[d], sem))
        copies.append(pltpu.async_copy(w1_hbm.at[i1b[k]], gb_v.at[d], sem))
        copies.append(pltpu.async_copy(s_hbm.at[i0b[k]], gc_v.at[d], sem))
        copies.append(pltpu.async_copy(s_hbm.at[i1b[k]], gd_v.at[d], sem))
    for c in copies:
        c.wait()
    pltpu.sync_copy(ga_v, ga_hbm.at[pl.ds(base, 512)])
    pltpu.sync_copy(gb_v, gb_hbm.at[pl.ds(base, 512)])
    pltpu.sync_copy(gc_v, gc_hbm.at[pl.ds(base, 512)])
    pltpu.sync_copy(gd_v, gd_hbm.at[pl.ds(base, 512)])


def _k4_epilogue_body(ga_ref, gb_ref, gc_ref, gd_ref, sup_ref, out_ref):
    m0, m1 = _support_means(sup_ref[...])
    n2 = jnp.maximum(jnp.sqrt(jnp.sum(m0 * m0) + jnp.sum(m1 * m1)), _EPS)
    num = ga_ref[...] + gb_ref[...]
    n1 = jnp.maximum(jnp.sqrt(gc_ref[...] + gd_ref[...]), _EPS)
    out_ref[...] = num / (n1 * n2)


def kernel(query, support, symbol_emb):
    b = query.shape[0]
    v = symbol_emb.shape[0]
    tab_t = symbol_emb.T
    q0 = query[:, 0].astype(jnp.int32)
    q1 = query[:, 1].astype(jnp.int32)
    sidx = jnp.concatenate([support.reshape(-1).astype(jnp.int32),
                            jnp.zeros((6,), jnp.int32)])
    n_blk = (v + _BC - 1) // _BC
    n_col = n_blk * _BC

    sup = pl.pallas_call(
        _k1_support_body,
        grid_spec=pltpu.PrefetchScalarGridSpec(
            num_scalar_prefetch=1,
            grid=(16,),
            in_specs=[pl.BlockSpec((64, 128), lambda t, sref: (0, sref[t] // 128))],
            out_specs=pl.BlockSpec((64, 16), lambda t, sref: (0, 0)),
        ),
        out_shape=jax.ShapeDtypeStruct((64, 16), jnp.float32),
    )(sidx, tab_t)

    w0, w1, s = pl.pallas_call(
        _k2_dense_body,
        grid=(n_blk,),
        in_specs=[pl.BlockSpec((64, _BC), lambda i: (0, i)),
                  pl.BlockSpec((64, 16), lambda i: (0, 0))],
        out_specs=[pl.BlockSpec((_BC,), lambda i: (i,))] * 3,
        out_shape=[jax.ShapeDtypeStruct((n_col,), jnp.float32)] * 3,
    )(tab_t, sup)

    gather = functools.partial(
        pl.kernel,
        mesh=plsc.VectorSubcoreMesh(core_axis_name="c", subcore_axis_name="s",
                                    num_cores=2),
        out_type=(jax.ShapeDtypeStruct((b,), jnp.float32),) * 4,
        scratch_types=tuple(
            [pltpu.VMEM((_CHUNK,), jnp.int32)] * 8
            + [pltpu.VMEM((512,), jnp.float32)] * 4
            + [pltpu.SemaphoreType.DMA]),
    )(_k3_gather_body)
    ga, gb, gc, gd = gather(q0, q1, w0, w1, s)

    out2d = pl.pallas_call(
        _k4_epilogue_body,
        in_specs=[pl.BlockSpec((128, 128), lambda: (0, 0))] * 4
        + [pl.BlockSpec((64, 16), lambda: (0, 0))],
        out_specs=pl.BlockSpec((128, 128), lambda: (0, 0)),
        out_shape=jax.ShapeDtypeStruct((128, 128), jnp.float32),
    )(ga.reshape(128, 128), gb.reshape(128, 128), gc.reshape(128, 128),
      gd.reshape(128, 128), sup)
    return out2d.reshape(b)


# speedup vs baseline: 3.7589x; 1.1031x over previous
import functools

import jax
import jax.numpy as jnp
from jax import lax
from jax.experimental import pallas as pl
from jax.experimental.pallas import tpu as pltpu
from jax.experimental.pallas import tpu_sc as plsc

_NW = 32
_CHUNK = 128
_BC = 8192
_EPS = 1e-8


def _k1_support_body(sref, tab_ref, out_ref):
    t = pl.program_id(0)
    c = sref[t] % 128
    x = tab_ref[...]
    lane = lax.broadcasted_iota(jnp.int32, x.shape, 1)
    val = jnp.sum(jnp.where(lane == c, x, 0.0), axis=1,
                  keepdims=True)

    @pl.when(t == 0)
    def _():
        out_ref[...] = jnp.zeros_like(out_ref)

    col = lax.broadcasted_iota(jnp.int32, out_ref.shape, 1)
    out_ref[...] = out_ref[...] + jnp.where(col == t, val, 0.0)


def _support_means(sup):
    t = lax.broadcasted_iota(jnp.int32, sup.shape, 1)
    m0 = jnp.sum(jnp.where((t % 2 == 0) & (t < 10), sup, 0.0), axis=1,
                 keepdims=True) * 0.2
    m1 = jnp.sum(jnp.where((t % 2 == 1) & (t < 10), sup, 0.0), axis=1,
                 keepdims=True) * 0.2
    return m0, m1


def _k2_dense_body(tab_ref, sup_ref, w0_ref, w1_ref, s_ref):
    m0, m1 = _support_means(sup_ref[...])
    x = tab_ref[...]
    mm = jnp.concatenate([m0, m1], axis=1)
    dn = (((0,), (0,)), ((), ()))
    w = lax.dot_general(mm, x, dn,
                        preferred_element_type=jnp.float32)
    ones = jnp.ones((64, 1), jnp.float32)
    s = lax.dot_general(ones, x * x, dn,
                        preferred_element_type=jnp.float32)
    w0_ref[...] = w[0]
    w1_ref[...] = w[1]
    s_ref[...] = s[0]


def _k3_gather_body(q0_hbm, q1_hbm, w0_hbm, w1_hbm, s_hbm,
                    ga_hbm, gb_hbm, gc_hbm, gd_hbm, *scratch):
    i0b = scratch[0:4]
    i1b = scratch[4:8]
    ga_v, gb_v, gc_v, gd_v, sem = scratch[8:]
    wid = lax.axis_index("s") * 2 + lax.axis_index("c")
    base = wid * 512
    for k in range(4):
        pltpu.sync_copy(q0_hbm.at[pl.ds(base + k * _CHUNK, _CHUNK)], i0b[k])
        pltpu.sync_copy(q1_hbm.at[pl.ds(base + k * _CHUNK, _CHUNK)], i1b[k])
    copies = []
    for k in range(4):
        d = pl.ds(k * _CHUNK, _CHUNK)
        copies.append(pltpu.async_copy(w0_hbm.at[i0b[k]], ga_v.at[d], sem))
        copies.append(pltpu.async_copy(w1_hbm.at[i1b[k]], gb_v.at[d], sem))
        copies.append(pltpu.async_copy(s_hbm.at[i0b[k]], gc_v.at[d], sem))
        copies.append(pltpu.async_copy(s_hbm.at[i1b[k]], gd_v.at[d], sem))
    for c in copies:
        c.wait()
    pltpu.sync_copy(ga_v, ga_hbm.at[pl.ds(base, 512)])
    pltpu.sync_copy(gb_v, gb_hbm.at[pl.ds(base, 512)])
    pltpu.sync_copy(gc_v, gc_hbm.at[pl.ds(base, 512)])
    pltpu.sync_copy(gd_v, gd_hbm.at[pl.ds(base, 512)])


def _k4_epilogue_body(ga_ref, gb_ref, gc_ref, gd_ref, sup_ref, out_ref):
    m0, m1 = _support_means(sup_ref[...])
    n2 = jnp.maximum(jnp.sqrt(jnp.sum(m0 * m0) + jnp.sum(m1 * m1)), _EPS)
    num = ga_ref[...] + gb_ref[...]
    n1 = jnp.maximum(jnp.sqrt(gc_ref[...] + gd_ref[...]), _EPS)
    out_ref[...] = num / (n1 * n2)


def kernel(query, support, symbol_emb):
    b = query.shape[0]
    v = symbol_emb.shape[0]
    tab_t = symbol_emb.T
    q0 = query[:, 0].astype(jnp.int32)
    q1 = query[:, 1].astype(jnp.int32)
    sidx = jnp.concatenate([support.reshape(-1).astype(jnp.int32),
                            jnp.zeros((6,), jnp.int32)])
    n_blk = (v + _BC - 1) // _BC
    n_col = n_blk * _BC

    sup = pl.pallas_call(
        _k1_support_body,
        grid_spec=pltpu.PrefetchScalarGridSpec(
            num_scalar_prefetch=1,
            grid=(16,),
            in_specs=[pl.BlockSpec((64, 128), lambda t, sref: (0, sref[t] // 128))],
            out_specs=pl.BlockSpec((64, 16), lambda t, sref: (0, 0)),
        ),
        out_shape=jax.ShapeDtypeStruct((64, 16), jnp.float32),
    )(sidx, tab_t)

    w0, w1, s = pl.pallas_call(
        _k2_dense_body,
        grid=(n_blk,),
        in_specs=[pl.BlockSpec((64, _BC), lambda i: (0, i)),
                  pl.BlockSpec((64, 16), lambda i: (0, 0))],
        out_specs=[pl.BlockSpec((_BC,), lambda i: (i,))] * 3,
        out_shape=[jax.ShapeDtypeStruct((n_col,), jnp.float32)] * 3,
    )(tab_t, sup)

    gather = functools.partial(
        pl.kernel,
        mesh=plsc.VectorSubcoreMesh(core_axis_name="c", subcore_axis_name="s",
                                    num_cores=2),
        out_type=(jax.ShapeDtypeStruct((b,), jnp.float32),) * 4,
        scratch_types=tuple(
            [pltpu.VMEM((_CHUNK,), jnp.int32)] * 8
            + [pltpu.VMEM((512,), jnp.float32)] * 4
            + [pltpu.SemaphoreType.DMA]),
    )(_k3_gather_body)
    ga, gb, gc, gd = gather(q0, q1, w0, w1, s)

    out2d = pl.pallas_call(
        _k4_epilogue_body,
        in_specs=[pl.BlockSpec((128, 128), lambda: (0, 0))] * 4
        + [pl.BlockSpec((64, 16), lambda: (0, 0))],
        out_specs=pl.BlockSpec((128, 128), lambda: (0, 0)),
        out_shape=jax.ShapeDtypeStruct((128, 128), jnp.float32),
    )(ga.reshape(128, 128), gb.reshape(128, 128), gc.reshape(128, 128),
      gd.reshape(128, 128), sup)
    return out2d.reshape(b)


# speedup vs baseline: 4.8957x; 1.3024x over previous
import functools

import jax
import jax.numpy as jnp
from jax import lax
from jax.experimental import pallas as pl
from jax.experimental.pallas import tpu as pltpu
from jax.experimental.pallas import tpu_sc as plsc

_NW = 32
_CHUNK = 128
_BC = 16384
_EPS = 1e-8


def _k1_support_body(sref, tab_ref, out_ref):
    t = pl.program_id(0)
    c = sref[t] % 128
    x = tab_ref[...]
    lane = lax.broadcasted_iota(jnp.int32, x.shape, 1)
    val = jnp.sum(jnp.where(lane == c, x, 0.0), axis=1,
                  keepdims=True)

    @pl.when(t == 0)
    def _():
        out_ref[...] = jnp.zeros_like(out_ref)

    col = lax.broadcasted_iota(jnp.int32, out_ref.shape, 1)
    out_ref[...] = out_ref[...] + jnp.where(col == t, val, 0.0)


def _support_means(sup):
    t = lax.broadcasted_iota(jnp.int32, sup.shape, 1)
    m0 = jnp.sum(jnp.where((t % 2 == 0) & (t < 10), sup, 0.0), axis=1,
                 keepdims=True) * 0.2
    m1 = jnp.sum(jnp.where((t % 2 == 1) & (t < 10), sup, 0.0), axis=1,
                 keepdims=True) * 0.2
    return m0, m1


def _k2_dense_body(tab_ref, sup_ref, w0_ref, w1_ref, s_ref):
    m0, m1 = _support_means(sup_ref[...])
    x = tab_ref[...]
    mm = jnp.concatenate([m0, m1], axis=1)
    dn = (((0,), (0,)), ((), ()))
    w = lax.dot_general(mm, x, dn,
                        preferred_element_type=jnp.float32)
    ones = jnp.ones((64, 1), jnp.float32)
    s = lax.dot_general(ones, x * x, dn,
                        preferred_element_type=jnp.float32)
    w0_ref[...] = w[0]
    w1_ref[...] = w[1]
    s_ref[...] = s[0]


def _k3_gather_body(q0_hbm, q1_hbm, w0_hbm, w1_hbm, s_hbm,
                    ga_hbm, gb_hbm, gc_hbm, gd_hbm, *scratch):
    i0b = scratch[0:4]
    i1b = scratch[4:8]
    ga_v, gb_v, gc_v, gd_v, sem = scratch[8:]
    wid = lax.axis_index("s") * 2 + lax.axis_index("c")
    base = wid * 512
    for k in range(4):
        pltpu.sync_copy(q0_hbm.at[pl.ds(base + k * _CHUNK, _CHUNK)], i0b[k])
        pltpu.sync_copy(q1_hbm.at[pl.ds(base + k * _CHUNK, _CHUNK)], i1b[k])
    copies = []
    for k in range(4):
        d = pl.ds(k * _CHUNK, _CHUNK)
        copies.append(pltpu.async_copy(w0_hbm.at[i0b[k]], ga_v.at[d], sem))
        copies.append(pltpu.async_copy(w1_hbm.at[i1b[k]], gb_v.at[d], sem))
        copies.append(pltpu.async_copy(s_hbm.at[i0b[k]], gc_v.at[d], sem))
        copies.append(pltpu.async_copy(s_hbm.at[i1b[k]], gd_v.at[d], sem))
    for c in copies:
        c.wait()
    pltpu.sync_copy(ga_v, ga_hbm.at[pl.ds(base, 512)])
    pltpu.sync_copy(gb_v, gb_hbm.at[pl.ds(base, 512)])
    pltpu.sync_copy(gc_v, gc_hbm.at[pl.ds(base, 512)])
    pltpu.sync_copy(gd_v, gd_hbm.at[pl.ds(base, 512)])


def _k4_epilogue_body(ga_ref, gb_ref, gc_ref, gd_ref, sup_ref, out_ref):
    m0, m1 = _support_means(sup_ref[...])
    n2 = jnp.maximum(jnp.sqrt(jnp.sum(m0 * m0) + jnp.sum(m1 * m1)), _EPS)
    num = ga_ref[...] + gb_ref[...]
    n1 = jnp.maximum(jnp.sqrt(gc_ref[...] + gd_ref[...]), _EPS)
    out_ref[...] = num / (n1 * n2)


def kernel(query, support, symbol_emb):
    b = query.shape[0]
    v = symbol_emb.shape[0]
    tab_t = symbol_emb.T
    q0 = query[:, 0].astype(jnp.int32)
    q1 = query[:, 1].astype(jnp.int32)
    sidx = jnp.concatenate([support.reshape(-1).astype(jnp.int32),
                            jnp.zeros((6,), jnp.int32)])
    n_blk = (v + _BC - 1) // _BC
    n_col = n_blk * _BC

    sup = pl.pallas_call(
        _k1_support_body,
        grid_spec=pltpu.PrefetchScalarGridSpec(
            num_scalar_prefetch=1,
            grid=(16,),
            in_specs=[pl.BlockSpec((64, 128), lambda t, sref: (0, sref[t] // 128))],
            out_specs=pl.BlockSpec((64, 16), lambda t, sref: (0, 0)),
        ),
        out_shape=jax.ShapeDtypeStruct((64, 16), jnp.float32),
    )(sidx, tab_t)

    w0, w1, s = pl.pallas_call(
        _k2_dense_body,
        grid=(n_blk,),
        in_specs=[pl.BlockSpec((64, _BC), lambda i: (0, i)),
                  pl.BlockSpec((64, 16), lambda i: (0, 0))],
        out_specs=[pl.BlockSpec((_BC,), lambda i: (i,))] * 3,
        out_shape=[jax.ShapeDtypeStruct((n_col,), jnp.float32)] * 3,
        compiler_params=pltpu.CompilerParams(
            dimension_semantics=("parallel",)),
    )(tab_t, sup)

    gather = functools.partial(
        pl.kernel,
        mesh=plsc.VectorSubcoreMesh(core_axis_name="c", subcore_axis_name="s",
                                    num_cores=2),
        out_type=(jax.ShapeDtypeStruct((b,), jnp.float32),) * 4,
        scratch_types=tuple(
            [pltpu.VMEM((_CHUNK,), jnp.int32)] * 8
            + [pltpu.VMEM((512,), jnp.float32)] * 4
            + [pltpu.SemaphoreType.DMA]),
    )(_k3_gather_body)
    ga, gb, gc, gd = gather(q0, q1, w0, w1, s)

    out2d = pl.pallas_call(
        _k4_epilogue_body,
        in_specs=[pl.BlockSpec((128, 128), lambda: (0, 0))] * 4
        + [pl.BlockSpec((64, 16), lambda: (0, 0))],
        out_specs=pl.BlockSpec((128, 128), lambda: (0, 0)),
        out_shape=jax.ShapeDtypeStruct((128, 128), jnp.float32),
    )(ga.reshape(128, 128), gb.reshape(128, 128), gc.reshape(128, 128),
      gd.reshape(128, 128), sup)
    return out2d.reshape(b)


# speedup vs baseline: 5.6998x; 1.1643x over previous
import functools

import jax
import jax.numpy as jnp
from jax import lax
from jax.experimental import pallas as pl
from jax.experimental.pallas import tpu as pltpu
from jax.experimental.pallas import tpu_sc as plsc

_NW = 32
_CHUNK = 128
_BC = 32768
_EPS = 1e-8


def _k1_support_body(sref, tab_ref, out_ref):
    t = pl.program_id(0)
    c = sref[t] % 128
    x = tab_ref[...]
    lane = lax.broadcasted_iota(jnp.int32, x.shape, 1)
    val = jnp.sum(jnp.where(lane == c, x, 0.0), axis=1,
                  keepdims=True)

    @pl.when(t == 0)
    def _():
        out_ref[...] = jnp.zeros_like(out_ref)

    col = lax.broadcasted_iota(jnp.int32, out_ref.shape, 1)
    out_ref[...] = out_ref[...] + jnp.where(col == t, val, 0.0)


def _support_means(sup):
    t = lax.broadcasted_iota(jnp.int32, sup.shape, 1)
    m0 = jnp.sum(jnp.where((t % 2 == 0) & (t < 10), sup, 0.0), axis=1,
                 keepdims=True) * 0.2
    m1 = jnp.sum(jnp.where((t % 2 == 1) & (t < 10), sup, 0.0), axis=1,
                 keepdims=True) * 0.2
    return m0, m1


def _k2_dense_body(tab_ref, sup_ref, w0_ref, w1_ref, s_ref):
    m0, m1 = _support_means(sup_ref[...])
    x = tab_ref[...]
    mm = jnp.concatenate([m0, m1], axis=1)
    dn = (((0,), (0,)), ((), ()))
    w = lax.dot_general(mm, x, dn,
                        preferred_element_type=jnp.float32)
    ones = jnp.ones((64, 1), jnp.float32)
    s = lax.dot_general(ones, x * x, dn,
                        preferred_element_type=jnp.float32)
    w0_ref[...] = w[0]
    w1_ref[...] = w[1]
    s_ref[...] = s[0]


def _k3_gather_body(q0_hbm, q1_hbm, w0_hbm, w1_hbm, s_hbm,
                    ga_hbm, gb_hbm, gc_hbm, gd_hbm, *scratch):
    i0b = scratch[0:4]
    i1b = scratch[4:8]
    ga_v, gb_v, gc_v, gd_v, sem = scratch[8:]
    wid = lax.axis_index("s") * 2 + lax.axis_index("c")
    base = wid * 512
    for k in range(4):
        pltpu.sync_copy(q0_hbm.at[pl.ds(base + k * _CHUNK, _CHUNK)], i0b[k])
        pltpu.sync_copy(q1_hbm.at[pl.ds(base + k * _CHUNK, _CHUNK)], i1b[k])
    copies = []
    for k in range(4):
        d = pl.ds(k * _CHUNK, _CHUNK)
        copies.append(pltpu.async_copy(w0_hbm.at[i0b[k]], ga_v.at[d], sem))
        copies.append(pltpu.async_copy(w1_hbm.at[i1b[k]], gb_v.at[d], sem))
        copies.append(pltpu.async_copy(s_hbm.at[i0b[k]], gc_v.at[d], sem))
        copies.append(pltpu.async_copy(s_hbm.at[i1b[k]], gd_v.at[d], sem))
    for c in copies:
        c.wait()
    pltpu.sync_copy(ga_v, ga_hbm.at[pl.ds(base, 512)])
    pltpu.sync_copy(gb_v, gb_hbm.at[pl.ds(base, 512)])
    pltpu.sync_copy(gc_v, gc_hbm.at[pl.ds(base, 512)])
    pltpu.sync_copy(gd_v, gd_hbm.at[pl.ds(base, 512)])


def _k4_epilogue_body(ga_ref, gb_ref, gc_ref, gd_ref, sup_ref, out_ref):
    m0, m1 = _support_means(sup_ref[...])
    n2 = jnp.maximum(jnp.sqrt(jnp.sum(m0 * m0) + jnp.sum(m1 * m1)), _EPS)
    num = ga_ref[...] + gb_ref[...]
    n1 = jnp.maximum(jnp.sqrt(gc_ref[...] + gd_ref[...]), _EPS)
    out_ref[...] = num / (n1 * n2)


def kernel(query, support, symbol_emb):
    b = query.shape[0]
    v = symbol_emb.shape[0]
    tab_t = symbol_emb.T
    q0 = query[:, 0].astype(jnp.int32)
    q1 = query[:, 1].astype(jnp.int32)
    sidx = jnp.concatenate([support.reshape(-1).astype(jnp.int32),
                            jnp.zeros((6,), jnp.int32)])
    n_blk = (v + _BC - 1) // _BC
    n_col = n_blk * _BC

    sup = pl.pallas_call(
        _k1_support_body,
        grid_spec=pltpu.PrefetchScalarGridSpec(
            num_scalar_prefetch=1,
            grid=(16,),
            in_specs=[pl.BlockSpec((64, 128), lambda t, sref: (0, sref[t] // 128))],
            out_specs=pl.BlockSpec((64, 16), lambda t, sref: (0, 0)),
        ),
        out_shape=jax.ShapeDtypeStruct((64, 16), jnp.float32),
    )(sidx, tab_t)

    w0, w1, s = pl.pallas_call(
        _k2_dense_body,
        grid=(n_blk,),
        in_specs=[pl.BlockSpec((64, _BC), lambda i: (0, i)),
                  pl.BlockSpec((64, 16), lambda i: (0, 0))],
        out_specs=[pl.BlockSpec((_BC,), lambda i: (i,))] * 3,
        out_shape=[jax.ShapeDtypeStruct((n_col,), jnp.float32)] * 3,
        compiler_params=pltpu.CompilerParams(
            dimension_semantics=("parallel",)),
    )(tab_t, sup)

    gather = functools.partial(
        pl.kernel,
        mesh=plsc.VectorSubcoreMesh(core_axis_name="c", subcore_axis_name="s",
                                    num_cores=2),
        out_type=(jax.ShapeDtypeStruct((b,), jnp.float32),) * 4,
        scratch_types=tuple(
            [pltpu.VMEM((_CHUNK,), jnp.int32)] * 8
            + [pltpu.VMEM((512,), jnp.float32)] * 4
            + [pltpu.SemaphoreType.DMA]),
    )(_k3_gather_body)
    ga, gb, gc, gd = gather(q0, q1, w0, w1, s)

    out2d = pl.pallas_call(
        _k4_epilogue_body,
        in_specs=[pl.BlockSpec((128, 128), lambda: (0, 0))] * 4
        + [pl.BlockSpec((64, 16), lambda: (0, 0))],
        out_specs=pl.BlockSpec((128, 128), lambda: (0, 0)),
        out_shape=jax.ShapeDtypeStruct((128, 128), jnp.float32),
    )(ga.reshape(128, 128), gb.reshape(128, 128), gc.reshape(128, 128),
      gd.reshape(128, 128), sup)
    return out2d.reshape(b)


# speedup vs baseline: 5.8223x; 1.0215x over previous
import functools

import jax
import jax.numpy as jnp
from jax import lax
from jax.experimental import pallas as pl
from jax.experimental.pallas import tpu as pltpu
from jax.experimental.pallas import tpu_sc as plsc

_NW = 32
_CHUNK = 128
_BC = 65536
_EPS = 1e-8


def _k1_support_body(sidx_ref, tab_ref, out_ref, buf, sem):
    copies = []
    for t in range(16):
        tile = pl.multiple_of((sidx_ref[t] // 128) * 128, 128)
        copies.append(pltpu.async_copy(
            tab_ref.at[:, pl.ds(tile, 128)], buf.at[t], sem))
    for c in copies:
        c.wait()
    lane = lax.broadcasted_iota(jnp.int32, (64, 128), 1)
    cols = []
    for t in range(16):
        c = sidx_ref[t] % 128
        cols.append(jnp.sum(jnp.where(lane == c, buf[t], 0.0), axis=1,
                            keepdims=True))
    out_ref[...] = jnp.concatenate(cols, axis=1)


def _support_means(sup):
    t = lax.broadcasted_iota(jnp.int32, sup.shape, 1)
    m0 = jnp.sum(jnp.where((t % 2 == 0) & (t < 10), sup, 0.0), axis=1,
                 keepdims=True) * 0.2
    m1 = jnp.sum(jnp.where((t % 2 == 1) & (t < 10), sup, 0.0), axis=1,
                 keepdims=True) * 0.2
    return m0, m1


def _k2_dense_body(tab_ref, sup_ref, w0_ref, w1_ref, s_ref):
    m0, m1 = _support_means(sup_ref[...])
    x = tab_ref[...]
    mm = jnp.concatenate([m0, m1], axis=1)
    dn = (((0,), (0,)), ((), ()))
    w = lax.dot_general(mm, x, dn,
                        preferred_element_type=jnp.float32)
    ones = jnp.ones((64, 1), jnp.float32)
    s = lax.dot_general(ones, x * x, dn,
                        preferred_element_type=jnp.float32)
    w0_ref[...] = w[0]
    w1_ref[...] = w[1]
    s_ref[...] = s[0]


def _k3_gather_body(q0_hbm, q1_hbm, w0_hbm, w1_hbm, s_hbm,
                    ga_hbm, gb_hbm, gc_hbm, gd_hbm, *scratch):
    i0b = scratch[0:4]
    i1b = scratch[4:8]
    ga_v, gb_v, gc_v, gd_v, sem = scratch[8:]
    wid = lax.axis_index("s") * 2 + lax.axis_index("c")
    base = wid * 512
    for k in range(4):
        pltpu.sync_copy(q0_hbm.at[pl.ds(base + k * _CHUNK, _CHUNK)], i0b[k])
        pltpu.sync_copy(q1_hbm.at[pl.ds(base + k * _CHUNK, _CHUNK)], i1b[k])
    copies = []
    for k in range(4):
        d = pl.ds(k * _CHUNK, _CHUNK)
        copies.append(pltpu.async_copy(w0_hbm.at[i0b[k]], ga_v.at[d], sem))
        copies.append(pltpu.async_copy(w1_hbm.at[i1b[k]], gb_v.at[d], sem))
        copies.append(pltpu.async_copy(s_hbm.at[i0b[k]], gc_v.at[d], sem))
        copies.append(pltpu.async_copy(s_hbm.at[i1b[k]], gd_v.at[d], sem))
    for c in copies:
        c.wait()
    pltpu.sync_copy(ga_v, ga_hbm.at[pl.ds(base, 512)])
    pltpu.sync_copy(gb_v, gb_hbm.at[pl.ds(base, 512)])
    pltpu.sync_copy(gc_v, gc_hbm.at[pl.ds(base, 512)])
    pltpu.sync_copy(gd_v, gd_hbm.at[pl.ds(base, 512)])


def _k4_epilogue_body(ga_ref, gb_ref, gc_ref, gd_ref, sup_ref, out_ref):
    m0, m1 = _support_means(sup_ref[...])
    n2 = jnp.maximum(jnp.sqrt(jnp.sum(m0 * m0) + jnp.sum(m1 * m1)), _EPS)
    num = ga_ref[...] + gb_ref[...]
    n1 = jnp.maximum(jnp.sqrt(gc_ref[...] + gd_ref[...]), _EPS)
    out_ref[...] = num / (n1 * n2)


def kernel(query, support, symbol_emb):
    b = query.shape[0]
    v = symbol_emb.shape[0]
    tab_t = symbol_emb.T
    q0 = query[:, 0].astype(jnp.int32)
    q1 = query[:, 1].astype(jnp.int32)
    sidx = jnp.concatenate([support.reshape(-1).astype(jnp.int32),
                            jnp.zeros((6,), jnp.int32)])
    n_blk = (v + _BC - 1) // _BC
    n_col = n_blk * _BC

    sup = pl.pallas_call(
        _k1_support_body,
        in_specs=[pl.BlockSpec(memory_space=pltpu.SMEM),
                  pl.BlockSpec(memory_space=pl.ANY)],
        out_specs=pl.BlockSpec((64, 16), lambda: (0, 0)),
        out_shape=jax.ShapeDtypeStruct((64, 16), jnp.float32),
        scratch_shapes=[pltpu.VMEM((16, 64, 128), jnp.float32),
                        pltpu.SemaphoreType.DMA],
    )(sidx, tab_t)

    w0, w1, s = pl.pallas_call(
        _k2_dense_body,
        grid=(n_blk,),
        in_specs=[pl.BlockSpec((64, _BC), lambda i: (0, i)),
                  pl.BlockSpec((64, 16), lambda i: (0, 0))],
        out_specs=[pl.BlockSpec((_BC,), lambda i: (i,))] * 3,
        out_shape=[jax.ShapeDtypeStruct((n_col,), jnp.float32)] * 3,
        compiler_params=pltpu.CompilerParams(
            dimension_semantics=("parallel",)),
    )(tab_t, sup)

    gather = functools.partial(
        pl.kernel,
        mesh=plsc.VectorSubcoreMesh(core_axis_name="c", subcore_axis_name="s",
                                    num_cores=2),
        out_type=(jax.ShapeDtypeStruct((b,), jnp.float32),) * 4,
        scratch_types=tuple(
            [pltpu.VMEM((_CHUNK,), jnp.int32)] * 8
            + [pltpu.VMEM((512,), jnp.float32)] * 4
            + [pltpu.SemaphoreType.DMA]),
    )(_k3_gather_body)
    ga, gb, gc, gd = gather(q0, q1, w0, w1, s)

    out2d = pl.pallas_call(
        _k4_epilogue_body,
        in_specs=[pl.BlockSpec((128, 128), lambda: (0, 0))] * 4
        + [pl.BlockSpec((64, 16), lambda: (0, 0))],
        out_specs=pl.BlockSpec((128, 128), lambda: (0, 0)),
        out_shape=jax.ShapeDtypeStruct((128, 128), jnp.float32),
    )(ga.reshape(128, 128), gb.reshape(128, 128), gc.reshape(128, 128),
      gd.reshape(128, 128), sup)
    return out2d.reshape(b)


# speedup vs baseline: 5.8643x; 1.0072x over previous
import functools

import jax
import jax.numpy as jnp
from jax import lax
from jax.experimental import pallas as pl
from jax.experimental.pallas import tpu as pltpu
from jax.experimental.pallas import tpu_sc as plsc

_NW = 32
_CHUNK = 128
_BC = 65536
_EPS = 1e-8


def _k1_support_body(sidx_ref, tab_ref, out_ref, invn2_ref, buf, sem):
    copies = []
    for t in range(16):
        tile = pl.multiple_of((sidx_ref[t] // 128) * 128, 128)
        copies.append(pltpu.async_copy(
            tab_ref.at[:, pl.ds(tile, 128)], buf.at[t], sem))
    for c in copies:
        c.wait()
    lane = lax.broadcasted_iota(jnp.int32, (64, 128), 1)
    cols = []
    for t in range(16):
        c = sidx_ref[t] % 128
        cols.append(jnp.sum(jnp.where(lane == c, buf[t], 0.0), axis=1,
                            keepdims=True))
    sup = jnp.concatenate(cols, axis=1)
    out_ref[...] = sup
    m0, m1 = _support_means(sup)
    n2 = jnp.maximum(jnp.sqrt(jnp.sum(m0 * m0) + jnp.sum(m1 * m1)), _EPS)
    invn2_ref[...] = jnp.full((16,), 1.0 / n2, jnp.float32)


def _support_means(sup):
    t = lax.broadcasted_iota(jnp.int32, sup.shape, 1)
    m0 = jnp.sum(jnp.where((t % 2 == 0) & (t < 10), sup, 0.0), axis=1,
                 keepdims=True) * 0.2
    m1 = jnp.sum(jnp.where((t % 2 == 1) & (t < 10), sup, 0.0), axis=1,
                 keepdims=True) * 0.2
    return m0, m1


def _k2_dense_body(tab_ref, sup_ref, w0_ref, w1_ref, s_ref):
    m0, m1 = _support_means(sup_ref[...])
    x = tab_ref[...]
    mm = jnp.concatenate([m0, m1], axis=1)
    dn = (((0,), (0,)), ((), ()))
    w = lax.dot_general(mm, x, dn,
                        preferred_element_type=jnp.float32)
    ones = jnp.ones((64, 1), jnp.float32)
    s = lax.dot_general(ones, x * x, dn,
                        preferred_element_type=jnp.float32)
    w0_ref[...] = w[0]
    w1_ref[...] = w[1]
    s_ref[...] = s[0]


def _k3_gather_body(q0_hbm, q1_hbm, w0_hbm, w1_hbm, s_hbm, invn2_hbm,
                    out_hbm, *scratch):
    i0b = scratch[0:4]
    i1b = scratch[4:8]
    ga_v, gb_v, gc_v, gd_v, out_v, inv_v, sem = scratch[8:]
    wid = lax.axis_index("s") * 2 + lax.axis_index("c")
    base = wid * 512
    pltpu.sync_copy(invn2_hbm, inv_v)
    for k in range(4):
        pltpu.sync_copy(q0_hbm.at[pl.ds(base + k * _CHUNK, _CHUNK)], i0b[k])
        pltpu.sync_copy(q1_hbm.at[pl.ds(base + k * _CHUNK, _CHUNK)], i1b[k])
    copies = []
    for k in range(4):
        d = pl.ds(k * _CHUNK, _CHUNK)
        copies.append(pltpu.async_copy(w0_hbm.at[i0b[k]], ga_v.at[d], sem))
        copies.append(pltpu.async_copy(w1_hbm.at[i1b[k]], gb_v.at[d], sem))
        copies.append(pltpu.async_copy(s_hbm.at[i0b[k]], gc_v.at[d], sem))
        copies.append(pltpu.async_copy(s_hbm.at[i1b[k]], gd_v.at[d], sem))
    for c in copies:
        c.wait()
    inv_n2 = inv_v[...]
    for i in range(512 // 16):
        d = pl.ds(i * 16, 16)
        num = ga_v[d] + gb_v[d]
        sq = gc_v[d] + gd_v[d]
        bits = lax.bitcast_convert_type(sq, jnp.int32)
        y = lax.bitcast_convert_type(
            0x5F3759DF - lax.shift_right_logical(bits, 1), jnp.float32)
        for _ in range(3):
            y = y * (1.5 - 0.5 * sq * y * y)
        out_v[d] = num * y * inv_n2
    pltpu.sync_copy(out_v, out_hbm.at[pl.ds(base, 512)])


def kernel(query, support, symbol_emb):
    b = query.shape[0]
    v = symbol_emb.shape[0]
    tab_t = symbol_emb.T
    q0 = query[:, 0].astype(jnp.int32)
    q1 = query[:, 1].astype(jnp.int32)
    sidx = jnp.concatenate([support.reshape(-1).astype(jnp.int32),
                            jnp.zeros((6,), jnp.int32)])
    n_blk = (v + _BC - 1) // _BC
    n_col = n_blk * _BC

    sup, invn2 = pl.pallas_call(
        _k1_support_body,
        in_specs=[pl.BlockSpec(memory_space=pltpu.SMEM),
                  pl.BlockSpec(memory_space=pl.ANY)],
        out_specs=[pl.BlockSpec((64, 16), lambda: (0, 0)),
                   pl.BlockSpec((16,), lambda: (0,))],
        out_shape=[jax.ShapeDtypeStruct((64, 16), jnp.float32),
                   jax.ShapeDtypeStruct((16,), jnp.float32)],
        scratch_shapes=[pltpu.VMEM((16, 64, 128), jnp.float32),
                        pltpu.SemaphoreType.DMA],
    )(sidx, tab_t)

    w0, w1, s = pl.pallas_call(
        _k2_dense_body,
        grid=(n_blk,),
        in_specs=[pl.BlockSpec((64, _BC), lambda i: (0, i)),
                  pl.BlockSpec((64, 16), lambda i: (0, 0))],
        out_specs=[pl.BlockSpec((_BC,), lambda i: (i,))] * 3,
        out_shape=[jax.ShapeDtypeStruct((n_col,), jnp.float32)] * 3,
        compiler_params=pltpu.CompilerParams(
            dimension_semantics=("parallel",)),
    )(tab_t, sup)

    gather = functools.partial(
        pl.kernel,
        mesh=plsc.VectorSubcoreMesh(core_axis_name="c", subcore_axis_name="s",
                                    num_cores=2),
        out_type=jax.ShapeDtypeStruct((b,), jnp.float32),
        scratch_types=tuple(
            [pltpu.VMEM((_CHUNK,), jnp.int32)] * 8
            + [pltpu.VMEM((512,), jnp.float32)] * 5
            + [pltpu.VMEM((16,), jnp.float32)]
            + [pltpu.SemaphoreType.DMA]),
    )(_k3_gather_body)
    return gather(q0, q1, w0, w1, s, invn2)
